# Initial kernel scaffold; baseline (speedup 1.0000x reference)
import jax
import jax.numpy as jnp
from jax.experimental import pallas as pl


def kernel(x, edge_attr, u, edge_index, We1, be1, We2, be2, Wn1, bn1, Wn2, bn2, Wg1, bg1, Wg2, bg2):
    raise NotImplementedError("write your pallas kernel here")


# speedup vs baseline: 1.7192x; 1.7192x over previous
import functools

import jax
import jax.numpy as jnp
from jax import lax
from jax.experimental import pallas as pl
from jax.experimental.pallas import tpu as pltpu
from jax.experimental.pallas import tpu_sc as plsc

_NC = 2
_NS = 16
_NW = _NC * _NS
_CHUNK = 80

_HIGH = lax.Precision.HIGHEST


def _sc_mesh():
    return plsc.VectorSubcoreMesh(core_axis_name="c", subcore_axis_name="s",
                                  num_cores=_NC, num_subcores=_NS)


@functools.lru_cache(maxsize=None)
def _make_gather(E, D):
    epw = E // _NW
    nch = epw // _CHUNK
    c = _CHUNK

    @functools.partial(
        pl.kernel,
        out_type=jax.ShapeDtypeStruct((E, D), jnp.float32),
        mesh=_sc_mesh(),
        scratch_types=[
            pltpu.VMEM((c,), jnp.int32),
            pltpu.VMEM((c,), jnp.int32),
            pltpu.VMEM((c, D), jnp.float32),
            pltpu.VMEM((c, D), jnp.float32),
            pltpu.SemaphoreType.DMA,
            pltpu.SemaphoreType.DMA,
        ],
        compiler_params=pltpu.CompilerParams(use_tc_tiling_on_sc=False),
    )
    def gather_kernel(ps_hbm, pd_hbm, src_hbm, dst_hbm, out_hbm,
                      idx_s, idx_d, buf_a, buf_b, sem_a, sem_b):
        wid = lax.axis_index("s") * _NC + lax.axis_index("c")
        base = wid * epw

        def chunk_body(j, carry):
            off = base + j * c
            pltpu.sync_copy(src_hbm.at[pl.ds(off, c)], idx_s)
            pltpu.sync_copy(dst_hbm.at[pl.ds(off, c)], idx_d)
            cp_a = pltpu.async_copy(ps_hbm.at[idx_s], buf_a, sem_a)
            cp_b = pltpu.async_copy(pd_hbm.at[idx_d], buf_b, sem_b)
            cp_a.wait()
            cp_b.wait()

            def add_row(i, _):
                for k in range(D // 16):
                    sl = pl.ds(k * 16, 16)
                    buf_a[i, sl] = buf_a[i, sl] + buf_b[i, sl]
                return 0

            lax.fori_loop(0, c, add_row, 0)
            pltpu.sync_copy(buf_a, out_hbm.at[pl.ds(off, c)])
            return 0

        lax.fori_loop(0, nch, chunk_body, 0)

    return gather_kernel


@functools.lru_cache(maxsize=None)
def _make_scatter(E, N, F):
    epw = E // _NW
    nch = epw // _CHUNK
    c = _CHUNK
    rpt = N // _NS

    @functools.partial(
        pl.kernel,
        out_type=jax.ShapeDtypeStruct((2 * N, F), jnp.float32),
        mesh=_sc_mesh(),
        scratch_types=[
            pltpu.VMEM((c,), jnp.int32),
            pltpu.VMEM((c, F), jnp.float32),
            pltpu.VMEM((rpt, F), jnp.float32),
            pltpu.VMEM_SHARED((N, F), jnp.float32),
        ],
        compiler_params=pltpu.CompilerParams(use_tc_tiling_on_sc=False),
    )
    def scatter_kernel(e_hbm, dst_hbm, out_hbm, idx_d, rows, obuf, agg_sh):
        cid = lax.axis_index("c")
        sid = lax.axis_index("s")
        wid = sid * _NC + cid
        base = wid * epw

        def zero_row(i, _):
            obuf[i, :] = jnp.zeros((F,), jnp.float32)
            return 0

        lax.fori_loop(0, rpt, zero_row, 0)
        pltpu.sync_copy(obuf, agg_sh.at[pl.ds(sid * rpt, rpt)])
        plsc.subcore_barrier()

        def chunk_body(j, carry):
            off = base + j * c
            pltpu.sync_copy(dst_hbm.at[pl.ds(off, c)], idx_d)
            pltpu.sync_copy(e_hbm.at[pl.ds(off, c)], rows)
            pltpu.sync_copy(rows, agg_sh.at[idx_d], add=True)
            return 0

        lax.fori_loop(0, nch, chunk_body, 0)
        plsc.subcore_barrier()
        pltpu.sync_copy(agg_sh.at[pl.ds(sid * rpt, rpt)], obuf)
        pltpu.sync_copy(obuf, out_hbm.at[pl.ds(cid * N + sid * rpt, rpt)])

    return scatter_kernel


def _proj_tc(x, a_src, a_dst):
    N, NN = x.shape
    D = a_src.shape[1]
    bn = 2000
    grid = (N // bn,)

    def body(x_ref, ws_ref, wd_ref, ps_ref, pd_ref):
        xv = x_ref[...]
        ps_ref[...] = jnp.dot(xv, ws_ref[...], precision=_HIGH,
                              preferred_element_type=jnp.float32)
        pd_ref[...] = jnp.dot(xv, wd_ref[...], precision=_HIGH,
                              preferred_element_type=jnp.float32)

    return pl.pallas_call(
        body,
        grid=grid,
        in_specs=[
            pl.BlockSpec((bn, NN), lambda j: (j, 0)),
            pl.BlockSpec((NN, D), lambda j: (0, 0)),
            pl.BlockSpec((NN, D), lambda j: (0, 0)),
        ],
        out_specs=[
            pl.BlockSpec((bn, D), lambda j: (j, 0)),
            pl.BlockSpec((bn, D), lambda j: (j, 0)),
        ],
        out_shape=[
            jax.ShapeDtypeStruct((N, D), jnp.float32),
            jax.ShapeDtypeStruct((N, D), jnp.float32),
        ],
    )(x, a_src, a_dst)


def _edge_tc(G, ea, u2, a_e, a_u, be1_2, we2, be2_2, last):
    E, D = G.shape
    F = ea.shape[1]
    be = 8000
    grid = (E // be,)

    def body(g_ref, ea_ref, u_ref, ae_ref, au_ref, b1_ref, w2_ref, b2_ref,
             out_ref, sum_ref):
        j = pl.program_id(0)
        cu = jnp.dot(u_ref[...], au_ref[...], precision=_HIGH,
                     preferred_element_type=jnp.float32) + b1_ref[...]
        h = jnp.maximum(
            g_ref[...]
            + jnp.dot(ea_ref[...], ae_ref[...], precision=_HIGH,
                      preferred_element_type=jnp.float32)
            + cu, 0.0)
        out = jnp.dot(h, w2_ref[...], precision=_HIGH,
                      preferred_element_type=jnp.float32) + b2_ref[...]
        if not last:
            out = jnp.maximum(out, 0.0)
        out_ref[...] = out

        @pl.when(j == 0)
        def _():
            sum_ref[...] = jnp.zeros_like(sum_ref)

        sum_ref[...] += jnp.sum(out, axis=0, keepdims=True)

    return pl.pallas_call(
        body,
        grid=grid,
        in_specs=[
            pl.BlockSpec((be, D), lambda j: (j, 0)),
            pl.BlockSpec((be, F), lambda j: (j, 0)),
            pl.BlockSpec((1, 4), lambda j: (0, 0)),
            pl.BlockSpec((F, D), lambda j: (0, 0)),
            pl.BlockSpec((4, D), lambda j: (0, 0)),
            pl.BlockSpec((1, D), lambda j: (0, 0)),
            pl.BlockSpec((D, F), lambda j: (0, 0)),
            pl.BlockSpec((1, F), lambda j: (0, 0)),
        ],
        out_specs=[
            pl.BlockSpec((be, F), lambda j: (j, 0)),
            pl.BlockSpec((1, F), lambda j: (0, 0)),
        ],
        out_shape=[
            jax.ShapeDtypeStruct((E, F), jnp.float32),
            jax.ShapeDtypeStruct((1, F), jnp.float32),
        ],
    )(G, ea, u2, a_e, a_u, be1_2, we2, be2_2)


def _node_tc(x, agg2, u2, wn_x, wn_a, wn_u, bn1_2, wn2, bn2_2,
             sum_e, wg_x, wg_e, wg_u, bg1_2, wg2, bg2_2,
             a_src_n, a_dst_n, last, n_edges):
    N, NN = x.shape
    F = wn_a.shape[0]
    H = wn_x.shape[1]
    bn = 2000
    grid = (N // bn,)
    nsteps = N // bn
    D = None if last else a_src_n.shape[1]

    def body(x_ref, agga_ref, aggb_ref, u_ref, wnx_ref, wna_ref, wnu_ref,
             b1_ref, wn2_ref, b2_ref, sume_ref, wgx_ref, wge_ref, wgu_ref,
             bg1_ref, wg2_ref, bg2_ref, *rest):
        if last:
            (asrc_ref, adst_ref, x_out, u_out, acc) = (None, None) + rest
        else:
            (asrc_ref, adst_ref, x_out, ps_out, pd_out, u_out, acc) = rest
        j = pl.program_id(0)
        cun = jnp.dot(u_ref[...], wnu_ref[...], precision=_HIGH,
                      preferred_element_type=jnp.float32) + b1_ref[...]
        agg = agga_ref[...] + aggb_ref[...]
        h = jnp.maximum(
            jnp.dot(x_ref[...], wnx_ref[...], precision=_HIGH,
                    preferred_element_type=jnp.float32)
            + jnp.dot(agg, wna_ref[...], precision=_HIGH,
                      preferred_element_type=jnp.float32)
            + cun, 0.0)
        xn = jnp.dot(h, wn2_ref[...], precision=_HIGH,
                     preferred_element_type=jnp.float32) + b2_ref[...]
        if not last:
            xn = jnp.maximum(xn, 0.0)
        x_out[...] = xn
        if not last:
            ps_out[...] = jnp.dot(xn, asrc_ref[...], precision=_HIGH,
                                  preferred_element_type=jnp.float32)
            pd_out[...] = jnp.dot(xn, adst_ref[...], precision=_HIGH,
                                  preferred_element_type=jnp.float32)

        @pl.when(j == 0)
        def _():
            acc[...] = jnp.zeros_like(acc)

        acc[...] += jnp.sum(xn, axis=0, keepdims=True)

        @pl.when(j == nsteps - 1)
        def _():
            mx = acc[...] * (1.0 / N)
            me = sume_ref[...] * (1.0 / n_edges)
            hg = jnp.maximum(
                jnp.dot(mx, wgx_ref[...], precision=_HIGH,
                        preferred_element_type=jnp.float32)
                + jnp.dot(me, wge_ref[...], precision=_HIGH,
                          preferred_element_type=jnp.float32)
                + jnp.dot(u_ref[...], wgu_ref[...], precision=_HIGH,
                          preferred_element_type=jnp.float32)
                + bg1_ref[...], 0.0)
            un = jnp.dot(hg, wg2_ref[...], precision=_HIGH,
                         preferred_element_type=jnp.float32) + bg2_ref[...]
            if not last:
                un = jnp.maximum(un, 0.0)
            u_out[...] = un

    HG = wg2.shape[0]
    GN = wg2.shape[1]
    in_specs = [
        pl.BlockSpec((bn, NN), lambda j: (j, 0)),
        pl.BlockSpec((bn, F), lambda j: (j, 0)),
        pl.BlockSpec((bn, F), lambda j: (j + nsteps, 0)),
        pl.BlockSpec((1, GN), lambda j: (0, 0)),
        pl.BlockSpec((NN, H), lambda j: (0, 0)),
        pl.BlockSpec((F, H), lambda j: (0, 0)),
        pl.BlockSpec((GN, H), lambda j: (0, 0)),
        pl.BlockSpec((1, H), lambda j: (0, 0)),
        pl.BlockSpec((H, NN), lambda j: (0, 0)),
        pl.BlockSpec((1, NN), lambda j: (0, 0)),
        pl.BlockSpec((1, F), lambda j: (0, 0)),
        pl.BlockSpec((NN, HG), lambda j: (0, 0)),
        pl.BlockSpec((F, HG), lambda j: (0, 0)),
        pl.BlockSpec((GN, HG), lambda j: (0, 0)),
        pl.BlockSpec((1, HG), lambda j: (0, 0)),
        pl.BlockSpec((HG, GN), lambda j: (0, 0)),
        pl.BlockSpec((1, GN), lambda j: (0, 0)),
    ]
    args = [x, agg2, agg2, u2, wn_x, wn_a, wn_u, bn1_2, wn2, bn2_2,
            sum_e, wg_x, wg_e, wg_u, bg1_2, wg2, bg2_2]
    out_specs = [pl.BlockSpec((bn, NN), lambda j: (j, 0))]
    out_shape = [jax.ShapeDtypeStruct((N, NN), jnp.float32)]
    if not last:
        in_specs += [
            pl.BlockSpec((NN, D), lambda j: (0, 0)),
            pl.BlockSpec((NN, D), lambda j: (0, 0)),
        ]
        args += [a_src_n, a_dst_n]
        out_specs += [
            pl.BlockSpec((bn, D), lambda j: (j, 0)),
            pl.BlockSpec((bn, D), lambda j: (j, 0)),
        ]
        out_shape += [
            jax.ShapeDtypeStruct((N, D), jnp.float32),
            jax.ShapeDtypeStruct((N, D), jnp.float32),
        ]
    out_specs += [pl.BlockSpec((1, GN), lambda j: (0, 0))]
    out_shape += [jax.ShapeDtypeStruct((1, GN), jnp.float32)]

    return pl.pallas_call(
        body,
        grid=grid,
        in_specs=in_specs,
        out_specs=out_specs,
        out_shape=out_shape,
        scratch_shapes=[pltpu.VMEM((1, NN), jnp.float32)],
    )(*args)


def kernel(x, edge_attr, u, edge_index,
           We1, be1, We2, be2,
           Wn1, bn1, Wn2, bn2,
           Wg1, bg1, Wg2, bg2):
    N, NN = x.shape
    E, EN = edge_attr.shape
    GN = u.shape[0]
    L = We1.shape[0]
    HE = We1.shape[2]

    src = edge_index[0]
    dst = edge_index[1]
    u2 = u.reshape(1, GN)

    gather = _make_gather(E, HE)
    scatter = _make_scatter(E, N, EN)

    ps, pd = _proj_tc(x, We1[0][:NN], We1[0][NN:2 * NN])

    for l in range(L):
        last = (l == L - 1)
        a_e = We1[l][2 * NN:2 * NN + EN]
        a_u = We1[l][2 * NN + EN:]
        g = gather(ps, pd, src, dst)
        edge_attr, sum_e = _edge_tc(g, edge_attr, u2, a_e, a_u,
                                    be1[l].reshape(1, HE), We2[l],
                                    be2[l].reshape(1, EN), last)
        agg2 = scatter(edge_attr, dst)
        wn_x = Wn1[l][:NN]
        wn_a = Wn1[l][NN:NN + EN]
        wn_u = Wn1[l][NN + EN:]
        wg_x = Wg1[l][:NN]
        wg_e = Wg1[l][NN:NN + EN]
        wg_u = Wg1[l][NN + EN:]
        if last:
            x, u2 = _node_tc(x, agg2, u2, wn_x, wn_a, wn_u,
                             bn1[l].reshape(1, -1), Wn2[l],
                             bn2[l].reshape(1, NN), sum_e,
                             wg_x, wg_e, wg_u, bg1[l].reshape(1, -1),
                             Wg2[l], bg2[l].reshape(1, GN),
                             None, None, last, E)
        else:
            a_src_n = We1[l + 1][:NN]
            a_dst_n = We1[l + 1][NN:2 * NN]
            x, ps, pd, u2 = _node_tc(x, agg2, u2, wn_x, wn_a, wn_u,
                                     bn1[l].reshape(1, -1), Wn2[l],
                                     bn2[l].reshape(1, NN), sum_e,
                                     wg_x, wg_e, wg_u, bg1[l].reshape(1, -1),
                                     Wg2[l], bg2[l].reshape(1, GN),
                                     a_src_n, a_dst_n, last, E)

    return x, edge_attr, u2.reshape(GN)


# speedup vs baseline: 2.4924x; 1.4497x over previous
import functools

import jax
import jax.numpy as jnp
from jax import lax
from jax.experimental import pallas as pl
from jax.experimental.pallas import tpu as pltpu
from jax.experimental.pallas import tpu_sc as plsc

_NC = 2
_NS = 16
_NW = _NC * _NS
_CHUNK = 80

_HIGH = lax.Precision.HIGHEST
_EDGEP = lax.Precision.DEFAULT


def _sc_mesh():
    return plsc.VectorSubcoreMesh(core_axis_name="c", subcore_axis_name="s",
                                  num_cores=_NC, num_subcores=_NS)


@functools.lru_cache(maxsize=None)
def _make_gather(E, D):
    epw = E // _NW
    nch = epw // _CHUNK
    c = _CHUNK

    @functools.partial(
        pl.kernel,
        out_type=jax.ShapeDtypeStruct((E, D), jnp.float32),
        mesh=_sc_mesh(),
        scratch_types=[
            pltpu.VMEM((c,), jnp.int32),
            pltpu.VMEM((c,), jnp.int32),
            pltpu.VMEM((c, D), jnp.float32),
            pltpu.VMEM((c, D), jnp.float32),
            pltpu.SemaphoreType.DMA,
            pltpu.SemaphoreType.DMA,
        ],
        compiler_params=pltpu.CompilerParams(use_tc_tiling_on_sc=False),
    )
    def gather_kernel(ps_hbm, pd_hbm, src_hbm, dst_hbm, out_hbm,
                      idx_s, idx_d, buf_a, buf_b, sem_a, sem_b):
        wid = lax.axis_index("s") * _NC + lax.axis_index("c")
        base = wid * epw

        def chunk_body(j, carry):
            off = base + j * c
            pltpu.sync_copy(src_hbm.at[pl.ds(off, c)], idx_s)
            pltpu.sync_copy(dst_hbm.at[pl.ds(off, c)], idx_d)
            cp_a = pltpu.async_copy(ps_hbm.at[idx_s], buf_a, sem_a)
            cp_b = pltpu.async_copy(pd_hbm.at[idx_d], buf_b, sem_b)
            cp_a.wait()
            cp_b.wait()

            def add_row(i, _):
                for k in range(D // 16):
                    sl = pl.ds(k * 16, 16)
                    buf_a[i, sl] = buf_a[i, sl] + buf_b[i, sl]
                return 0

            lax.fori_loop(0, c, add_row, 0)
            pltpu.sync_copy(buf_a, out_hbm.at[pl.ds(off, c)])
            return 0

        lax.fori_loop(0, nch, chunk_body, 0)

    return gather_kernel


@functools.lru_cache(maxsize=None)
def _make_scatter(E, N, F):
    epw = E // _NW
    nch = epw // _CHUNK
    c = _CHUNK
    rpt = N // _NS

    @functools.partial(
        pl.kernel,
        out_type=jax.ShapeDtypeStruct((2 * N, F), jnp.float32),
        mesh=_sc_mesh(),
        scratch_types=[
            pltpu.VMEM((c,), jnp.int32),
            pltpu.VMEM((c, F), jnp.float32),
            pltpu.VMEM((rpt, F), jnp.float32),
            pltpu.VMEM_SHARED((N, F), jnp.float32),
        ],
        compiler_params=pltpu.CompilerParams(use_tc_tiling_on_sc=False),
    )
    def scatter_kernel(e_hbm, dst_hbm, out_hbm, idx_d, rows, obuf, agg_sh):
        cid = lax.axis_index("c")
        sid = lax.axis_index("s")
        wid = sid * _NC + cid
        base = wid * epw

        def zero_row(i, _):
            obuf[i, :] = jnp.zeros((F,), jnp.float32)
            return 0

        lax.fori_loop(0, rpt, zero_row, 0)
        pltpu.sync_copy(obuf, agg_sh.at[pl.ds(sid * rpt, rpt)])
        plsc.subcore_barrier()

        def chunk_body(j, carry):
            off = base + j * c
            pltpu.sync_copy(dst_hbm.at[pl.ds(off, c)], idx_d)
            pltpu.sync_copy(e_hbm.at[pl.ds(off, c)], rows)
            pltpu.sync_copy(rows, agg_sh.at[idx_d], add=True)
            return 0

        lax.fori_loop(0, nch, chunk_body, 0)
        plsc.subcore_barrier()
        pltpu.sync_copy(agg_sh.at[pl.ds(sid * rpt, rpt)], obuf)
        pltpu.sync_copy(obuf, out_hbm.at[pl.ds(cid * N + sid * rpt, rpt)])

    return scatter_kernel


def _proj_tc(x, a_src, a_dst):
    N, NN = x.shape
    D = a_src.shape[1]
    bn = 2000
    grid = (N // bn,)

    def body(x_ref, ws_ref, wd_ref, ps_ref, pd_ref):
        xv = x_ref[...]
        ps_ref[...] = jnp.dot(xv, ws_ref[...], precision=_EDGEP,
                              preferred_element_type=jnp.float32)
        pd_ref[...] = jnp.dot(xv, wd_ref[...], precision=_EDGEP,
                              preferred_element_type=jnp.float32)

    return pl.pallas_call(
        body,
        grid=grid,
        in_specs=[
            pl.BlockSpec((bn, NN), lambda j: (j, 0)),
            pl.BlockSpec((NN, D), lambda j: (0, 0)),
            pl.BlockSpec((NN, D), lambda j: (0, 0)),
        ],
        out_specs=[
            pl.BlockSpec((bn, D), lambda j: (j, 0)),
            pl.BlockSpec((bn, D), lambda j: (j, 0)),
        ],
        out_shape=[
            jax.ShapeDtypeStruct((N, D), jnp.float32),
            jax.ShapeDtypeStruct((N, D), jnp.float32),
        ],
    )(x, a_src, a_dst)


def _edge_tc(G, ea, u2, a_e, a_u, be1_2, we2, be2_2, last):
    E, D = G.shape
    F = ea.shape[1]
    be = 8000
    grid = (E // be,)

    def body(g_ref, ea_ref, u_ref, ae_ref, au_ref, b1_ref, w2_ref, b2_ref,
             out_ref, sum_ref):
        j = pl.program_id(0)
        cu = jnp.dot(u_ref[...], au_ref[...], precision=_EDGEP,
                     preferred_element_type=jnp.float32) + b1_ref[...]
        h = jnp.maximum(
            g_ref[...]
            + jnp.dot(ea_ref[...], ae_ref[...], precision=_EDGEP,
                      preferred_element_type=jnp.float32)
            + cu, 0.0)
        out = jnp.dot(h, w2_ref[...], precision=_EDGEP,
                      preferred_element_type=jnp.float32) + b2_ref[...]
        if not last:
            out = jnp.maximum(out, 0.0)
        out_ref[...] = out

        @pl.when(j == 0)
        def _():
            sum_ref[...] = jnp.zeros_like(sum_ref)

        sum_ref[...] += jnp.sum(out, axis=0, keepdims=True)

    return pl.pallas_call(
        body,
        grid=grid,
        in_specs=[
            pl.BlockSpec((be, D), lambda j: (j, 0)),
            pl.BlockSpec((be, F), lambda j: (j, 0)),
            pl.BlockSpec((1, 4), lambda j: (0, 0)),
            pl.BlockSpec((F, D), lambda j: (0, 0)),
            pl.BlockSpec((4, D), lambda j: (0, 0)),
            pl.BlockSpec((1, D), lambda j: (0, 0)),
            pl.BlockSpec((D, F), lambda j: (0, 0)),
            pl.BlockSpec((1, F), lambda j: (0, 0)),
        ],
        out_specs=[
            pl.BlockSpec((be, F), lambda j: (j, 0)),
            pl.BlockSpec((1, F), lambda j: (0, 0)),
        ],
        out_shape=[
            jax.ShapeDtypeStruct((E, F), jnp.float32),
            jax.ShapeDtypeStruct((1, F), jnp.float32),
        ],
    )(G, ea, u2, a_e, a_u, be1_2, we2, be2_2)


def _node_tc(x, agg2, u2, wn_x, wn_a, wn_u, bn1_2, wn2, bn2_2,
             sum_e, wg_x, wg_e, wg_u, bg1_2, wg2, bg2_2,
             a_src_n, a_dst_n, last, n_edges):
    N, NN = x.shape
    F = wn_a.shape[0]
    H = wn_x.shape[1]
    bn = 2000
    grid = (N // bn,)
    nsteps = N // bn
    D = None if last else a_src_n.shape[1]

    def body(x_ref, agga_ref, aggb_ref, u_ref, wnx_ref, wna_ref, wnu_ref,
             b1_ref, wn2_ref, b2_ref, sume_ref, wgx_ref, wge_ref, wgu_ref,
             bg1_ref, wg2_ref, bg2_ref, *rest):
        if last:
            (asrc_ref, adst_ref, x_out, u_out, acc) = (None, None) + rest
        else:
            (asrc_ref, adst_ref, x_out, ps_out, pd_out, u_out, acc) = rest
        j = pl.program_id(0)
        cun = jnp.dot(u_ref[...], wnu_ref[...], precision=_EDGEP,
                      preferred_element_type=jnp.float32) + b1_ref[...]
        agg = agga_ref[...] + aggb_ref[...]
        h = jnp.maximum(
            jnp.dot(x_ref[...], wnx_ref[...], precision=_EDGEP,
                    preferred_element_type=jnp.float32)
            + jnp.dot(agg, wna_ref[...], precision=_EDGEP,
                      preferred_element_type=jnp.float32)
            + cun, 0.0)
        xn = jnp.dot(h, wn2_ref[...], precision=_EDGEP,
                     preferred_element_type=jnp.float32) + b2_ref[...]
        if not last:
            xn = jnp.maximum(xn, 0.0)
        x_out[...] = xn
        if not last:
            ps_out[...] = jnp.dot(xn, asrc_ref[...], precision=_EDGEP,
                                  preferred_element_type=jnp.float32)
            pd_out[...] = jnp.dot(xn, adst_ref[...], precision=_EDGEP,
                                  preferred_element_type=jnp.float32)

        @pl.when(j == 0)
        def _():
            acc[...] = jnp.zeros_like(acc)

        acc[...] += jnp.sum(xn, axis=0, keepdims=True)

        @pl.when(j == nsteps - 1)
        def _():
            mx = acc[...] * (1.0 / N)
            me = sume_ref[...] * (1.0 / n_edges)
            hg = jnp.maximum(
                jnp.dot(mx, wgx_ref[...], precision=_EDGEP,
                        preferred_element_type=jnp.float32)
                + jnp.dot(me, wge_ref[...], precision=_EDGEP,
                          preferred_element_type=jnp.float32)
                + jnp.dot(u_ref[...], wgu_ref[...], precision=_EDGEP,
                          preferred_element_type=jnp.float32)
                + bg1_ref[...], 0.0)
            un = jnp.dot(hg, wg2_ref[...], precision=_EDGEP,
                         preferred_element_type=jnp.float32) + bg2_ref[...]
            if not last:
                un = jnp.maximum(un, 0.0)
            u_out[...] = un

    HG = wg2.shape[0]
    GN = wg2.shape[1]
    in_specs = [
        pl.BlockSpec((bn, NN), lambda j: (j, 0)),
        pl.BlockSpec((bn, F), lambda j: (j, 0)),
        pl.BlockSpec((bn, F), lambda j: (j + nsteps, 0)),
        pl.BlockSpec((1, GN), lambda j: (0, 0)),
        pl.BlockSpec((NN, H), lambda j: (0, 0)),
        pl.BlockSpec((F, H), lambda j: (0, 0)),
        pl.BlockSpec((GN, H), lambda j: (0, 0)),
        pl.BlockSpec((1, H), lambda j: (0, 0)),
        pl.BlockSpec((H, NN), lambda j: (0, 0)),
        pl.BlockSpec((1, NN), lambda j: (0, 0)),
        pl.BlockSpec((1, F), lambda j: (0, 0)),
        pl.BlockSpec((NN, HG), lambda j: (0, 0)),
        pl.BlockSpec((F, HG), lambda j: (0, 0)),
        pl.BlockSpec((GN, HG), lambda j: (0, 0)),
        pl.BlockSpec((1, HG), lambda j: (0, 0)),
        pl.BlockSpec((HG, GN), lambda j: (0, 0)),
        pl.BlockSpec((1, GN), lambda j: (0, 0)),
    ]
    args = [x, agg2, agg2, u2, wn_x, wn_a, wn_u, bn1_2, wn2, bn2_2,
            sum_e, wg_x, wg_e, wg_u, bg1_2, wg2, bg2_2]
    out_specs = [pl.BlockSpec((bn, NN), lambda j: (j, 0))]
    out_shape = [jax.ShapeDtypeStruct((N, NN), jnp.float32)]
    if not last:
        in_specs += [
            pl.BlockSpec((NN, D), lambda j: (0, 0)),
            pl.BlockSpec((NN, D), lambda j: (0, 0)),
        ]
        args += [a_src_n, a_dst_n]
        out_specs += [
            pl.BlockSpec((bn, D), lambda j: (j, 0)),
            pl.BlockSpec((bn, D), lambda j: (j, 0)),
        ]
        out_shape += [
            jax.ShapeDtypeStruct((N, D), jnp.float32),
            jax.ShapeDtypeStruct((N, D), jnp.float32),
        ]
    out_specs += [pl.BlockSpec((1, GN), lambda j: (0, 0))]
    out_shape += [jax.ShapeDtypeStruct((1, GN), jnp.float32)]

    return pl.pallas_call(
        body,
        grid=grid,
        in_specs=in_specs,
        out_specs=out_specs,
        out_shape=out_shape,
        scratch_shapes=[pltpu.VMEM((1, NN), jnp.float32)],
    )(*args)


def kernel(x, edge_attr, u, edge_index,
           We1, be1, We2, be2,
           Wn1, bn1, Wn2, bn2,
           Wg1, bg1, Wg2, bg2):
    N, NN = x.shape
    E, EN = edge_attr.shape
    GN = u.shape[0]
    L = We1.shape[0]
    HE = We1.shape[2]

    src = edge_index[0]
    dst = edge_index[1]
    u2 = u.reshape(1, GN)

    gather = _make_gather(E, HE)
    scatter = _make_scatter(E, N, EN)

    ps, pd = _proj_tc(x, We1[0][:NN], We1[0][NN:2 * NN])

    for l in range(L):
        last = (l == L - 1)
        a_e = We1[l][2 * NN:2 * NN + EN]
        a_u = We1[l][2 * NN + EN:]
        g = gather(ps, pd, src, dst)
        edge_attr, sum_e = _edge_tc(g, edge_attr, u2, a_e, a_u,
                                    be1[l].reshape(1, HE), We2[l],
                                    be2[l].reshape(1, EN), last)
        agg2 = scatter(edge_attr, dst)
        wn_x = Wn1[l][:NN]
        wn_a = Wn1[l][NN:NN + EN]
        wn_u = Wn1[l][NN + EN:]
        wg_x = Wg1[l][:NN]
        wg_e = Wg1[l][NN:NN + EN]
        wg_u = Wg1[l][NN + EN:]
        if last:
            x, u2 = _node_tc(x, agg2, u2, wn_x, wn_a, wn_u,
                             bn1[l].reshape(1, -1), Wn2[l],
                             bn2[l].reshape(1, NN), sum_e,
                             wg_x, wg_e, wg_u, bg1[l].reshape(1, -1),
                             Wg2[l], bg2[l].reshape(1, GN),
                             None, None, last, E)
        else:
            a_src_n = We1[l + 1][:NN]
            a_dst_n = We1[l + 1][NN:2 * NN]
            x, ps, pd, u2 = _node_tc(x, agg2, u2, wn_x, wn_a, wn_u,
                                     bn1[l].reshape(1, -1), Wn2[l],
                                     bn2[l].reshape(1, NN), sum_e,
                                     wg_x, wg_e, wg_u, bg1[l].reshape(1, -1),
                                     Wg2[l], bg2[l].reshape(1, GN),
                                     a_src_n, a_dst_n, last, E)

    return x, edge_attr, u2.reshape(GN)


# speedup vs baseline: 3.7933x; 1.5220x over previous
import functools

import jax
import jax.numpy as jnp
from jax import lax
from jax.experimental import pallas as pl
from jax.experimental.pallas import tpu as pltpu
from jax.experimental.pallas import tpu_sc as plsc

_NC = 2
_NS = 16
_NW = _NC * _NS
_CHUNK = 80

_HIGH = lax.Precision.HIGHEST
_EDGEP = lax.Precision.DEFAULT


def _sc_mesh():
    return plsc.VectorSubcoreMesh(core_axis_name="c", subcore_axis_name="s",
                                  num_cores=_NC, num_subcores=_NS)


@functools.lru_cache(maxsize=None)
def _make_gather(E, D):
    epw = E // _NW
    nch = epw // _CHUNK
    c = _CHUNK
    nbuf = 5
    nslot = 2 * nbuf
    assert nch >= nslot

    scratch = (
        [pltpu.VMEM((2, c), jnp.int32) for _ in range(nslot)]
        + [pltpu.VMEM((c, D), jnp.float32) for _ in range(3 * nbuf)]
        + [pltpu.SemaphoreType.DMA for _ in range(nslot + 3 * nbuf)]
    )

    @functools.partial(
        pl.kernel,
        out_type=jax.ShapeDtypeStruct((E, D), jnp.float32),
        mesh=_sc_mesh(),
        scratch_types=scratch,
        compiler_params=pltpu.CompilerParams(use_tc_tiling_on_sc=False),
    )
    def gather_kernel(ps_hbm, pd_hbm, idxp_hbm, out_hbm, *s):
        idxv = s[:nslot]
        bufa = s[nslot:nslot + nbuf]
        bufb = s[nslot + nbuf:nslot + 2 * nbuf]
        bufo = s[nslot + 2 * nbuf:nslot + 3 * nbuf]
        semi = s[nslot + 3 * nbuf:2 * nslot + 3 * nbuf]
        sema = s[2 * nslot + 3 * nbuf:2 * nslot + 4 * nbuf]
        semb = s[2 * nslot + 4 * nbuf:2 * nslot + 5 * nbuf]
        semo = s[2 * nslot + 5 * nbuf:2 * nslot + 6 * nbuf]
        wid = lax.axis_index("s") * _NC + lax.axis_index("c")
        qbase = wid * nch

        def d_idx(q, slot):
            return pltpu.make_async_copy(idxp_hbm.at[qbase + q],
                                         idxv[slot], semi[slot])

        def d_a(b, slot):
            return pltpu.make_async_copy(ps_hbm.at[idxv[slot].at[0]],
                                         bufa[b], sema[b])

        def d_b(b, slot):
            return pltpu.make_async_copy(pd_hbm.at[idxv[slot].at[1]],
                                         bufb[b], semb[b])

        def d_out(b, q):
            return pltpu.make_async_copy(
                bufo[b], out_hbm.at[pl.ds((qbase + q) * c, c)], semo[b])

        for s0 in range(nslot):
            d_idx(s0, s0).start()
        for b0 in range(nbuf):
            d_idx(b0, b0).wait()
            d_a(b0, b0).start()
            d_b(b0, b0).start()

        def sg_body(g2, carry):
            j0 = g2 * nslot
            for k in range(nslot):
                b = k % nbuf
                j = j0 + k

                @pl.when(j < nch)
                def _process():
                    d_a(b, k).wait()
                    d_b(b, k).wait()

                    @pl.when(j >= nbuf)
                    def _():
                        d_out(b, j).wait()

                    def add_row(i, _):
                        for t in range(D // 16):
                            sl = pl.ds(t * 16, 16)
                            bufo[b][i, sl] = bufa[b][i, sl] + bufb[b][i, sl]
                        return 0

                    lax.fori_loop(0, c, add_row, 0)
                    d_out(b, j).start()

                @pl.when(j + nslot < nch)
                def _refill_idx():
                    d_idx(j + nslot, k).start()

                slot5 = (k + nbuf) % nslot

                @pl.when(j + nbuf < nch)
                def _fire_next():
                    d_idx(j + nbuf, slot5).wait()
                    d_a(b, slot5).start()
                    d_b(b, slot5).start()

            return 0

        lax.fori_loop(0, (nch + nslot - 1) // nslot, sg_body, 0)
        for b0 in range(nbuf):
            d_out(b0, 0).wait()

    return gather_kernel


@functools.lru_cache(maxsize=None)
def _make_scatter(E, N, F):
    epw = E // _NW
    nch = epw // _CHUNK
    c = _CHUNK
    rpt = N // _NS

    nbuf = 5
    assert nch % nbuf == 0

    scratch = (
        [pltpu.VMEM((2, c), jnp.int32) for _ in range(nbuf)]
        + [pltpu.VMEM((c, F), jnp.float32) for _ in range(nbuf)]
        + [pltpu.VMEM((rpt, F), jnp.float32),
           pltpu.VMEM_SHARED((N, F), jnp.float32)]
        + [pltpu.SemaphoreType.DMA for _ in range(2 * nbuf)]
    )

    @functools.partial(
        pl.kernel,
        out_type=jax.ShapeDtypeStruct((2 * N, F), jnp.float32),
        mesh=_sc_mesh(),
        scratch_types=scratch,
        compiler_params=pltpu.CompilerParams(use_tc_tiling_on_sc=False),
    )
    def scatter_kernel(e_hbm, idxp_hbm, out_hbm, *s):
        idxv = s[:nbuf]
        rows = s[nbuf:2 * nbuf]
        obuf = s[2 * nbuf]
        agg_sh = s[2 * nbuf + 1]
        semi = s[2 * nbuf + 2:3 * nbuf + 2]
        semr = s[3 * nbuf + 2:4 * nbuf + 2]
        cid = lax.axis_index("c")
        sid = lax.axis_index("s")
        wid = sid * _NC + cid
        qbase = wid * nch

        def d_idx(q, b):
            return pltpu.make_async_copy(idxp_hbm.at[qbase + q],
                                         idxv[b], semi[b])

        def d_rows(q, b):
            return pltpu.make_async_copy(
                e_hbm.at[pl.ds((qbase + q) * c, c)], rows[b], semr[b])

        for b0 in range(nbuf):
            d_idx(b0, b0).start()
            d_rows(b0, b0).start()

        def zero_row(i, _):
            obuf[i, :] = jnp.zeros((F,), jnp.float32)
            return 0

        lax.fori_loop(0, rpt, zero_row, 0)
        pltpu.sync_copy(obuf, agg_sh.at[pl.ds(sid * rpt, rpt)])
        plsc.subcore_barrier()

        def grp_body(g, carry):
            for b in range(nbuf):
                j = g * nbuf + b
                d_idx(j, b).wait()
                d_rows(j, b).wait()
                pltpu.sync_copy(rows[b], agg_sh.at[idxv[b].at[1]], add=True)

                @pl.when(j + nbuf < nch)
                def _refill():
                    d_idx(j + nbuf, b).start()
                    d_rows(j + nbuf, b).start()

            return 0

        lax.fori_loop(0, nch // nbuf, grp_body, 0)
        plsc.subcore_barrier()
        pltpu.sync_copy(agg_sh.at[pl.ds(sid * rpt, rpt)], obuf)
        pltpu.sync_copy(obuf, out_hbm.at[pl.ds(cid * N + sid * rpt, rpt)])

    return scatter_kernel


def _proj_tc(x, a_src, a_dst):
    N, NN = x.shape
    D = a_src.shape[1]
    bn = 2000
    grid = (N // bn,)

    def body(x_ref, ws_ref, wd_ref, ps_ref, pd_ref):
        xv = x_ref[...]
        ps_ref[...] = jnp.dot(xv, ws_ref[...], precision=_EDGEP,
                              preferred_element_type=jnp.float32)
        pd_ref[...] = jnp.dot(xv, wd_ref[...], precision=_EDGEP,
                              preferred_element_type=jnp.float32)

    return pl.pallas_call(
        body,
        grid=grid,
        in_specs=[
            pl.BlockSpec((bn, NN), lambda j: (j, 0)),
            pl.BlockSpec((NN, D), lambda j: (0, 0)),
            pl.BlockSpec((NN, D), lambda j: (0, 0)),
        ],
        out_specs=[
            pl.BlockSpec((bn, D), lambda j: (j, 0)),
            pl.BlockSpec((bn, D), lambda j: (j, 0)),
        ],
        out_shape=[
            jax.ShapeDtypeStruct((N, D), jnp.float32),
            jax.ShapeDtypeStruct((N, D), jnp.float32),
        ],
    )(x, a_src, a_dst)


def _edge_tc(G, ea, u2, a_e, a_u, be1_2, we2, be2_2, last):
    E, D = G.shape
    F = ea.shape[1]
    be = 8000
    grid = (E // be,)

    def body(g_ref, ea_ref, u_ref, ae_ref, au_ref, b1_ref, w2_ref, b2_ref,
             out_ref, sum_ref):
        j = pl.program_id(0)
        cu = jnp.dot(u_ref[...], au_ref[...], precision=_EDGEP,
                     preferred_element_type=jnp.float32) + b1_ref[...]
        h = jnp.maximum(
            g_ref[...]
            + jnp.dot(ea_ref[...], ae_ref[...], precision=_EDGEP,
                      preferred_element_type=jnp.float32)
            + cu, 0.0)
        out = jnp.dot(h, w2_ref[...], precision=_EDGEP,
                      preferred_element_type=jnp.float32) + b2_ref[...]
        if not last:
            out = jnp.maximum(out, 0.0)
        out_ref[...] = out

        @pl.when(j == 0)
        def _():
            sum_ref[...] = jnp.zeros_like(sum_ref)

        sum_ref[...] += jnp.sum(out, axis=0, keepdims=True)

    return pl.pallas_call(
        body,
        grid=grid,
        in_specs=[
            pl.BlockSpec((be, D), lambda j: (j, 0)),
            pl.BlockSpec((be, F), lambda j: (j, 0)),
            pl.BlockSpec((1, 4), lambda j: (0, 0)),
            pl.BlockSpec((F, D), lambda j: (0, 0)),
            pl.BlockSpec((4, D), lambda j: (0, 0)),
            pl.BlockSpec((1, D), lambda j: (0, 0)),
            pl.BlockSpec((D, F), lambda j: (0, 0)),
            pl.BlockSpec((1, F), lambda j: (0, 0)),
        ],
        out_specs=[
            pl.BlockSpec((be, F), lambda j: (j, 0)),
            pl.BlockSpec((1, F), lambda j: (0, 0)),
        ],
        out_shape=[
            jax.ShapeDtypeStruct((E, F), jnp.float32),
            jax.ShapeDtypeStruct((1, F), jnp.float32),
        ],
    )(G, ea, u2, a_e, a_u, be1_2, we2, be2_2)


def _node_tc(x, agg2, u2, wn_x, wn_a, wn_u, bn1_2, wn2, bn2_2,
             sum_e, wg_x, wg_e, wg_u, bg1_2, wg2, bg2_2,
             a_src_n, a_dst_n, last, n_edges):
    N, NN = x.shape
    F = wn_a.shape[0]
    H = wn_x.shape[1]
    bn = 2000
    grid = (N // bn,)
    nsteps = N // bn
    D = None if last else a_src_n.shape[1]

    def body(x_ref, agga_ref, aggb_ref, u_ref, wnx_ref, wna_ref, wnu_ref,
             b1_ref, wn2_ref, b2_ref, sume_ref, wgx_ref, wge_ref, wgu_ref,
             bg1_ref, wg2_ref, bg2_ref, *rest):
        if last:
            (asrc_ref, adst_ref, x_out, u_out, acc) = (None, None) + rest
        else:
            (asrc_ref, adst_ref, x_out, ps_out, pd_out, u_out, acc) = rest
        j = pl.program_id(0)
        cun = jnp.dot(u_ref[...], wnu_ref[...], precision=_EDGEP,
                      preferred_element_type=jnp.float32) + b1_ref[...]
        agg = agga_ref[...] + aggb_ref[...]
        h = jnp.maximum(
            jnp.dot(x_ref[...], wnx_ref[...], precision=_EDGEP,
                    preferred_element_type=jnp.float32)
            + jnp.dot(agg, wna_ref[...], precision=_EDGEP,
                      preferred_element_type=jnp.float32)
            + cun, 0.0)
        xn = jnp.dot(h, wn2_ref[...], precision=_EDGEP,
                     preferred_element_type=jnp.float32) + b2_ref[...]
        if not last:
            xn = jnp.maximum(xn, 0.0)
        x_out[...] = xn
        if not last:
            ps_out[...] = jnp.dot(xn, asrc_ref[...], precision=_EDGEP,
                                  preferred_element_type=jnp.float32)
            pd_out[...] = jnp.dot(xn, adst_ref[...], precision=_EDGEP,
                                  preferred_element_type=jnp.float32)

        @pl.when(j == 0)
        def _():
            acc[...] = jnp.zeros_like(acc)

        acc[...] += jnp.sum(xn, axis=0, keepdims=True)

        @pl.when(j == nsteps - 1)
        def _():
            mx = acc[...] * (1.0 / N)
            me = sume_ref[...] * (1.0 / n_edges)
            hg = jnp.maximum(
                jnp.dot(mx, wgx_ref[...], precision=_EDGEP,
                        preferred_element_type=jnp.float32)
                + jnp.dot(me, wge_ref[...], precision=_EDGEP,
                          preferred_element_type=jnp.float32)
                + jnp.dot(u_ref[...], wgu_ref[...], precision=_EDGEP,
                          preferred_element_type=jnp.float32)
                + bg1_ref[...], 0.0)
            un = jnp.dot(hg, wg2_ref[...], precision=_EDGEP,
                         preferred_element_type=jnp.float32) + bg2_ref[...]
            if not last:
                un = jnp.maximum(un, 0.0)
            u_out[...] = un

    HG = wg2.shape[0]
    GN = wg2.shape[1]
    in_specs = [
        pl.BlockSpec((bn, NN), lambda j: (j, 0)),
        pl.BlockSpec((bn, F), lambda j: (j, 0)),
        pl.BlockSpec((bn, F), lambda j: (j + nsteps, 0)),
        pl.BlockSpec((1, GN), lambda j: (0, 0)),
        pl.BlockSpec((NN, H), lambda j: (0, 0)),
        pl.BlockSpec((F, H), lambda j: (0, 0)),
        pl.BlockSpec((GN, H), lambda j: (0, 0)),
        pl.BlockSpec((1, H), lambda j: (0, 0)),
        pl.BlockSpec((H, NN), lambda j: (0, 0)),
        pl.BlockSpec((1, NN), lambda j: (0, 0)),
        pl.BlockSpec((1, F), lambda j: (0, 0)),
        pl.BlockSpec((NN, HG), lambda j: (0, 0)),
        pl.BlockSpec((F, HG), lambda j: (0, 0)),
        pl.BlockSpec((GN, HG), lambda j: (0, 0)),
        pl.BlockSpec((1, HG), lambda j: (0, 0)),
        pl.BlockSpec((HG, GN), lambda j: (0, 0)),
        pl.BlockSpec((1, GN), lambda j: (0, 0)),
    ]
    args = [x, agg2, agg2, u2, wn_x, wn_a, wn_u, bn1_2, wn2, bn2_2,
            sum_e, wg_x, wg_e, wg_u, bg1_2, wg2, bg2_2]
    out_specs = [pl.BlockSpec((bn, NN), lambda j: (j, 0))]
    out_shape = [jax.ShapeDtypeStruct((N, NN), jnp.float32)]
    if not last:
        in_specs += [
            pl.BlockSpec((NN, D), lambda j: (0, 0)),
            pl.BlockSpec((NN, D), lambda j: (0, 0)),
        ]
        args += [a_src_n, a_dst_n]
        out_specs += [
            pl.BlockSpec((bn, D), lambda j: (j, 0)),
            pl.BlockSpec((bn, D), lambda j: (j, 0)),
        ]
        out_shape += [
            jax.ShapeDtypeStruct((N, D), jnp.float32),
            jax.ShapeDtypeStruct((N, D), jnp.float32),
        ]
    out_specs += [pl.BlockSpec((1, GN), lambda j: (0, 0))]
    out_shape += [jax.ShapeDtypeStruct((1, GN), jnp.float32)]

    return pl.pallas_call(
        body,
        grid=grid,
        in_specs=in_specs,
        out_specs=out_specs,
        out_shape=out_shape,
        scratch_shapes=[pltpu.VMEM((1, NN), jnp.float32)],
    )(*args)


def kernel(x, edge_attr, u, edge_index,
           We1, be1, We2, be2,
           Wn1, bn1, Wn2, bn2,
           Wg1, bg1, Wg2, bg2):
    N, NN = x.shape
    E, EN = edge_attr.shape
    GN = u.shape[0]
    L = We1.shape[0]
    HE = We1.shape[2]

    src = edge_index[0]
    dst = edge_index[1]
    idxp = jnp.stack([src.reshape(E // _CHUNK, _CHUNK),
                      dst.reshape(E // _CHUNK, _CHUNK)], axis=1)
    u2 = u.reshape(1, GN)

    gather = _make_gather(E, HE)
    scatter = _make_scatter(E, N, EN)

    ps, pd = _proj_tc(x, We1[0][:NN], We1[0][NN:2 * NN])

    for l in range(L):
        last = (l == L - 1)
        a_e = We1[l][2 * NN:2 * NN + EN]
        a_u = We1[l][2 * NN + EN:]
        g = gather(ps, pd, idxp)
        edge_attr, sum_e = _edge_tc(g, edge_attr, u2, a_e, a_u,
                                    be1[l].reshape(1, HE), We2[l],
                                    be2[l].reshape(1, EN), last)
        agg2 = scatter(edge_attr, idxp)
        wn_x = Wn1[l][:NN]
        wn_a = Wn1[l][NN:NN + EN]
        wn_u = Wn1[l][NN + EN:]
        wg_x = Wg1[l][:NN]
        wg_e = Wg1[l][NN:NN + EN]
        wg_u = Wg1[l][NN + EN:]
        if last:
            x, u2 = _node_tc(x, agg2, u2, wn_x, wn_a, wn_u,
                             bn1[l].reshape(1, -1), Wn2[l],
                             bn2[l].reshape(1, NN), sum_e,
                             wg_x, wg_e, wg_u, bg1[l].reshape(1, -1),
                             Wg2[l], bg2[l].reshape(1, GN),
                             None, None, last, E)
        else:
            a_src_n = We1[l + 1][:NN]
            a_dst_n = We1[l + 1][NN:2 * NN]
            x, ps, pd, u2 = _node_tc(x, agg2, u2, wn_x, wn_a, wn_u,
                                     bn1[l].reshape(1, -1), Wn2[l],
                                     bn2[l].reshape(1, NN), sum_e,
                                     wg_x, wg_e, wg_u, bg1[l].reshape(1, -1),
                                     Wg2[l], bg2[l].reshape(1, GN),
                                     a_src_n, a_dst_n, last, E)

    return x, edge_attr, u2.reshape(GN)


# speedup vs baseline: 4.8912x; 1.2894x over previous
import functools

import jax
import jax.numpy as jnp
from jax import lax
from jax.experimental import pallas as pl
from jax.experimental.pallas import tpu as pltpu
from jax.experimental.pallas import tpu_sc as plsc

_NC = 2
_NS = 16
_NW = _NC * _NS
_CHUNK = 80

_HIGH = lax.Precision.HIGHEST
_EDGEP = lax.Precision.DEFAULT


def _sc_mesh():
    return plsc.VectorSubcoreMesh(core_axis_name="c", subcore_axis_name="s",
                                  num_cores=_NC, num_subcores=_NS)


@functools.lru_cache(maxsize=None)
def _make_gather(E, D):
    epw = E // _NW
    nch = epw // _CHUNK
    c = _CHUNK
    nbuf = 5
    nslot = 2 * nbuf
    assert nch >= nslot

    scratch = (
        [pltpu.VMEM((2, c), jnp.int32) for _ in range(nslot)]
        + [pltpu.VMEM((c, D), jnp.float32) for _ in range(2 * nbuf)]
        + [pltpu.VMEM((c, 128), jnp.float32) for _ in range(nbuf)]
        + [pltpu.SemaphoreType.DMA for _ in range(nslot + 3 * nbuf)]
    )

    @functools.partial(
        pl.kernel,
        out_type=jax.ShapeDtypeStruct((E, 128), jnp.float32),
        mesh=_sc_mesh(),
        scratch_types=scratch,
        compiler_params=pltpu.CompilerParams(use_tc_tiling_on_sc=False),
    )
    def gather_kernel(ps_hbm, pd_hbm, idxp_hbm, out_hbm, *s):
        idxv = s[:nslot]
        bufa = s[nslot:nslot + nbuf]
        bufb = s[nslot + nbuf:nslot + 2 * nbuf]
        bufo = s[nslot + 2 * nbuf:nslot + 3 * nbuf]
        semi = s[nslot + 3 * nbuf:2 * nslot + 3 * nbuf]
        sema = s[2 * nslot + 3 * nbuf:2 * nslot + 4 * nbuf]
        semb = s[2 * nslot + 4 * nbuf:2 * nslot + 5 * nbuf]
        semo = s[2 * nslot + 5 * nbuf:2 * nslot + 6 * nbuf]
        wid = lax.axis_index("s") * _NC + lax.axis_index("c")
        qbase = wid * nch

        def d_idx(q, slot):
            return pltpu.make_async_copy(idxp_hbm.at[qbase + q],
                                         idxv[slot], semi[slot])

        def d_a(b, slot):
            return pltpu.make_async_copy(ps_hbm.at[idxv[slot].at[0]],
                                         bufa[b], sema[b])

        def d_b(b, slot):
            return pltpu.make_async_copy(pd_hbm.at[idxv[slot].at[1]],
                                         bufb[b], semb[b])

        def d_out(b, q):
            return pltpu.make_async_copy(
                bufo[b], out_hbm.at[pl.ds((qbase + q) * c, c)], semo[b])

        for s0 in range(nslot):
            d_idx(s0, s0).start()
        for b0 in range(nbuf):
            d_idx(b0, b0).wait()
            d_a(b0, b0).start()
            d_b(b0, b0).start()

        def sg_body(g2, carry):
            j0 = g2 * nslot
            for k in range(nslot):
                b = k % nbuf
                j = j0 + k

                @pl.when(j < nch)
                def _process():
                    d_a(b, k).wait()
                    d_b(b, k).wait()

                    @pl.when(j >= nbuf)
                    def _():
                        d_out(b, j).wait()

                    def add_row(i, _):
                        for t in range(D // 16):
                            sl = pl.ds(t * 16, 16)
                            bufo[b][i, sl] = bufa[b][i, sl] + bufb[b][i, sl]
                        return 0

                    lax.fori_loop(0, c, add_row, 0)
                    d_out(b, j).start()

                @pl.when(j + nslot < nch)
                def _refill_idx():
                    d_idx(j + nslot, k).start()

                slot5 = (k + nbuf) % nslot

                @pl.when(j + nbuf < nch)
                def _fire_next():
                    d_idx(j + nbuf, slot5).wait()
                    d_a(b, slot5).start()
                    d_b(b, slot5).start()

            return 0

        lax.fori_loop(0, (nch + nslot - 1) // nslot, sg_body, 0)
        for b0 in range(nbuf):
            d_out(b0, 0).wait()

    return gather_kernel


@functools.lru_cache(maxsize=None)
def _make_scatter(E, N, F):
    epw = E // _NW
    nch = epw // _CHUNK
    c = _CHUNK
    rpt = N // _NS

    nbuf = 5
    assert nch % nbuf == 0

    scratch = (
        [pltpu.VMEM((2, c), jnp.int32) for _ in range(nbuf)]
        + [pltpu.VMEM((c, F), jnp.float32) for _ in range(nbuf)]
        + [pltpu.VMEM((rpt, F), jnp.float32),
           pltpu.VMEM_SHARED((N, F), jnp.float32)]
        + [pltpu.SemaphoreType.DMA for _ in range(2 * nbuf)]
    )

    @functools.partial(
        pl.kernel,
        out_type=jax.ShapeDtypeStruct((2 * N, F), jnp.float32),
        mesh=_sc_mesh(),
        scratch_types=scratch,
        compiler_params=pltpu.CompilerParams(use_tc_tiling_on_sc=False),
    )
    def scatter_kernel(e_hbm, idxp_hbm, out_hbm, *s):
        idxv = s[:nbuf]
        rows = s[nbuf:2 * nbuf]
        obuf = s[2 * nbuf]
        agg_sh = s[2 * nbuf + 1]
        semi = s[2 * nbuf + 2:3 * nbuf + 2]
        semr = s[3 * nbuf + 2:4 * nbuf + 2]
        cid = lax.axis_index("c")
        sid = lax.axis_index("s")
        wid = sid * _NC + cid
        qbase = wid * nch

        def d_idx(q, b):
            return pltpu.make_async_copy(idxp_hbm.at[qbase + q],
                                         idxv[b], semi[b])

        def d_rows(q, b):
            return pltpu.make_async_copy(
                e_hbm.at[pl.ds((qbase + q) * c, c), pl.ds(0, F)],
                rows[b], semr[b])

        for b0 in range(nbuf):
            d_idx(b0, b0).start()
            d_rows(b0, b0).start()

        def zero_row(i, _):
            obuf[i, :] = jnp.zeros((F,), jnp.float32)
            return 0

        lax.fori_loop(0, rpt, zero_row, 0)
        pltpu.sync_copy(obuf, agg_sh.at[pl.ds(sid * rpt, rpt)])
        plsc.subcore_barrier()

        def grp_body(g, carry):
            for b in range(nbuf):
                j = g * nbuf + b
                d_idx(j, b).wait()
                d_rows(j, b).wait()
                pltpu.sync_copy(rows[b], agg_sh.at[idxv[b].at[1]], add=True)

                @pl.when(j + nbuf < nch)
                def _refill():
                    d_idx(j + nbuf, b).start()
                    d_rows(j + nbuf, b).start()

            return 0

        lax.fori_loop(0, nch // nbuf, grp_body, 0)
        plsc.subcore_barrier()
        pltpu.sync_copy(agg_sh.at[pl.ds(sid * rpt, rpt)], obuf)
        pltpu.sync_copy(obuf, out_hbm.at[pl.ds(cid * N + sid * rpt, rpt)])

    return scatter_kernel


def _proj_tc(x, a_src, a_dst):
    N, NN = x.shape
    D = a_src.shape[1]
    bn = 2000
    grid = (N // bn,)

    def body(x_ref, ws_ref, wd_ref, ps_ref, pd_ref):
        xv = x_ref[...]
        ps_ref[...] = jnp.dot(xv, ws_ref[...], precision=_EDGEP,
                              preferred_element_type=jnp.float32)
        pd_ref[...] = jnp.dot(xv, wd_ref[...], precision=_EDGEP,
                              preferred_element_type=jnp.float32)

    return pl.pallas_call(
        body,
        grid=grid,
        in_specs=[
            pl.BlockSpec((bn, NN), lambda j: (j, 0)),
            pl.BlockSpec((NN, D), lambda j: (0, 0)),
            pl.BlockSpec((NN, D), lambda j: (0, 0)),
        ],
        out_specs=[
            pl.BlockSpec((bn, D), lambda j: (j, 0)),
            pl.BlockSpec((bn, D), lambda j: (j, 0)),
        ],
        out_shape=[
            jax.ShapeDtypeStruct((N, D), jnp.float32),
            jax.ShapeDtypeStruct((N, D), jnp.float32),
        ],
    )(x, a_src, a_dst)


def _edge_tc(G, ea, u2, a_e, a_u, be1_2, we2, be2_2, last):
    E = G.shape[0]
    F, D = a_e.shape
    ea_wide = ea.shape[1] != F
    be = 8000
    grid = (E // be,)

    def body(g_ref, ea_ref, u_ref, ae_ref, au_ref, b1_ref, w2_ref, b2_ref,
             out_ref, sum_ref):
        j = pl.program_id(0)
        cu = jnp.dot(u_ref[...], au_ref[...], precision=_EDGEP,
                     preferred_element_type=jnp.float32) + b1_ref[...]
        eav = ea_ref[:, :F] if ea_wide else ea_ref[...]
        h = jnp.maximum(
            g_ref[:, :D]
            + jnp.dot(eav, ae_ref[...], precision=_EDGEP,
                      preferred_element_type=jnp.float32)
            + cu, 0.0)
        out = jnp.dot(h, w2_ref[...], precision=_EDGEP,
                      preferred_element_type=jnp.float32) + b2_ref[...]
        if not last:
            out = jnp.maximum(out, 0.0)
        out_ref[:, :F] = out

        @pl.when(j == 0)
        def _():
            sum_ref[...] = jnp.zeros_like(sum_ref)

        sum_ref[...] += jnp.sum(out, axis=0, keepdims=True)

    return pl.pallas_call(
        body,
        grid=grid,
        in_specs=[
            pl.BlockSpec((be, 128), lambda j: (j, 0)),
            pl.BlockSpec((be, 128 if ea_wide else F), lambda j: (j, 0)),
            pl.BlockSpec((1, 4), lambda j: (0, 0)),
            pl.BlockSpec((F, D), lambda j: (0, 0)),
            pl.BlockSpec((4, D), lambda j: (0, 0)),
            pl.BlockSpec((1, D), lambda j: (0, 0)),
            pl.BlockSpec((D, F), lambda j: (0, 0)),
            pl.BlockSpec((1, F), lambda j: (0, 0)),
        ],
        out_specs=[
            pl.BlockSpec((be, 128), lambda j: (j, 0)),
            pl.BlockSpec((1, F), lambda j: (0, 0)),
        ],
        out_shape=[
            jax.ShapeDtypeStruct((E, 128), jnp.float32),
            jax.ShapeDtypeStruct((1, F), jnp.float32),
        ],
    )(G, ea, u2, a_e, a_u, be1_2, we2, be2_2)


def _node_tc(x, agg2, u2, wn_x, wn_a, wn_u, bn1_2, wn2, bn2_2,
             sum_e, wg_x, wg_e, wg_u, bg1_2, wg2, bg2_2,
             a_src_n, a_dst_n, last, n_edges):
    N, NN = x.shape
    F = wn_a.shape[0]
    H = wn_x.shape[1]
    bn = 2000
    grid = (N // bn,)
    nsteps = N // bn
    D = None if last else a_src_n.shape[1]

    def body(x_ref, agga_ref, aggb_ref, u_ref, wnx_ref, wna_ref, wnu_ref,
             b1_ref, wn2_ref, b2_ref, sume_ref, wgx_ref, wge_ref, wgu_ref,
             bg1_ref, wg2_ref, bg2_ref, *rest):
        if last:
            (asrc_ref, adst_ref, x_out, u_out, acc) = (None, None) + rest
        else:
            (asrc_ref, adst_ref, x_out, ps_out, pd_out, u_out, acc) = rest
        j = pl.program_id(0)
        cun = jnp.dot(u_ref[...], wnu_ref[...], precision=_EDGEP,
                      preferred_element_type=jnp.float32) + b1_ref[...]
        agg = agga_ref[...] + aggb_ref[...]
        h = jnp.maximum(
            jnp.dot(x_ref[...], wnx_ref[...], precision=_EDGEP,
                    preferred_element_type=jnp.float32)
            + jnp.dot(agg, wna_ref[...], precision=_EDGEP,
                      preferred_element_type=jnp.float32)
            + cun, 0.0)
        xn = jnp.dot(h, wn2_ref[...], precision=_EDGEP,
                     preferred_element_type=jnp.float32) + b2_ref[...]
        if not last:
            xn = jnp.maximum(xn, 0.0)
        x_out[...] = xn
        if not last:
            ps_out[...] = jnp.dot(xn, asrc_ref[...], precision=_EDGEP,
                                  preferred_element_type=jnp.float32)
            pd_out[...] = jnp.dot(xn, adst_ref[...], precision=_EDGEP,
                                  preferred_element_type=jnp.float32)

        @pl.when(j == 0)
        def _():
            acc[...] = jnp.zeros_like(acc)

        acc[...] += jnp.sum(xn, axis=0, keepdims=True)

        @pl.when(j == nsteps - 1)
        def _():
            mx = acc[...] * (1.0 / N)
            me = sume_ref[...] * (1.0 / n_edges)
            hg = jnp.maximum(
                jnp.dot(mx, wgx_ref[...], precision=_EDGEP,
                        preferred_element_type=jnp.float32)
                + jnp.dot(me, wge_ref[...], precision=_EDGEP,
                          preferred_element_type=jnp.float32)
                + jnp.dot(u_ref[...], wgu_ref[...], precision=_EDGEP,
                          preferred_element_type=jnp.float32)
                + bg1_ref[...], 0.0)
            un = jnp.dot(hg, wg2_ref[...], precision=_EDGEP,
                         preferred_element_type=jnp.float32) + bg2_ref[...]
            if not last:
                un = jnp.maximum(un, 0.0)
            u_out[...] = un

    HG = wg2.shape[0]
    GN = wg2.shape[1]
    in_specs = [
        pl.BlockSpec((bn, NN), lambda j: (j, 0)),
        pl.BlockSpec((bn, F), lambda j: (j, 0)),
        pl.BlockSpec((bn, F), lambda j: (j + nsteps, 0)),
        pl.BlockSpec((1, GN), lambda j: (0, 0)),
        pl.BlockSpec((NN, H), lambda j: (0, 0)),
        pl.BlockSpec((F, H), lambda j: (0, 0)),
        pl.BlockSpec((GN, H), lambda j: (0, 0)),
        pl.BlockSpec((1, H), lambda j: (0, 0)),
        pl.BlockSpec((H, NN), lambda j: (0, 0)),
        pl.BlockSpec((1, NN), lambda j: (0, 0)),
        pl.BlockSpec((1, F), lambda j: (0, 0)),
        pl.BlockSpec((NN, HG), lambda j: (0, 0)),
        pl.BlockSpec((F, HG), lambda j: (0, 0)),
        pl.BlockSpec((GN, HG), lambda j: (0, 0)),
        pl.BlockSpec((1, HG), lambda j: (0, 0)),
        pl.BlockSpec((HG, GN), lambda j: (0, 0)),
        pl.BlockSpec((1, GN), lambda j: (0, 0)),
    ]
    args = [x, agg2, agg2, u2, wn_x, wn_a, wn_u, bn1_2, wn2, bn2_2,
            sum_e, wg_x, wg_e, wg_u, bg1_2, wg2, bg2_2]
    out_specs = [pl.BlockSpec((bn, NN), lambda j: (j, 0))]
    out_shape = [jax.ShapeDtypeStruct((N, NN), jnp.float32)]
    if not last:
        in_specs += [
            pl.BlockSpec((NN, D), lambda j: (0, 0)),
            pl.BlockSpec((NN, D), lambda j: (0, 0)),
        ]
        args += [a_src_n, a_dst_n]
        out_specs += [
            pl.BlockSpec((bn, D), lambda j: (j, 0)),
            pl.BlockSpec((bn, D), lambda j: (j, 0)),
        ]
        out_shape += [
            jax.ShapeDtypeStruct((N, D), jnp.float32),
            jax.ShapeDtypeStruct((N, D), jnp.float32),
        ]
    out_specs += [pl.BlockSpec((1, GN), lambda j: (0, 0))]
    out_shape += [jax.ShapeDtypeStruct((1, GN), jnp.float32)]

    return pl.pallas_call(
        body,
        grid=grid,
        in_specs=in_specs,
        out_specs=out_specs,
        out_shape=out_shape,
        scratch_shapes=[pltpu.VMEM((1, NN), jnp.float32)],
    )(*args)


def kernel(x, edge_attr, u, edge_index,
           We1, be1, We2, be2,
           Wn1, bn1, Wn2, bn2,
           Wg1, bg1, Wg2, bg2):
    N, NN = x.shape
    E, EN = edge_attr.shape
    GN = u.shape[0]
    L = We1.shape[0]
    HE = We1.shape[2]

    src = edge_index[0]
    dst = edge_index[1]
    idxp = jnp.stack([src.reshape(E // _CHUNK, _CHUNK),
                      dst.reshape(E // _CHUNK, _CHUNK)], axis=1)
    u2 = u.reshape(1, GN)

    gather = _make_gather(E, HE)
    scatter = _make_scatter(E, N, EN)

    ps, pd = _proj_tc(x, We1[0][:NN], We1[0][NN:2 * NN])

    for l in range(L):
        last = (l == L - 1)
        a_e = We1[l][2 * NN:2 * NN + EN]
        a_u = We1[l][2 * NN + EN:]
        g = gather(ps, pd, idxp)
        edge_attr, sum_e = _edge_tc(g, edge_attr, u2, a_e, a_u,
                                    be1[l].reshape(1, HE), We2[l],
                                    be2[l].reshape(1, EN), last)
        agg2 = scatter(edge_attr, idxp)
        wn_x = Wn1[l][:NN]
        wn_a = Wn1[l][NN:NN + EN]
        wn_u = Wn1[l][NN + EN:]
        wg_x = Wg1[l][:NN]
        wg_e = Wg1[l][NN:NN + EN]
        wg_u = Wg1[l][NN + EN:]
        if last:
            x, u2 = _node_tc(x, agg2, u2, wn_x, wn_a, wn_u,
                             bn1[l].reshape(1, -1), Wn2[l],
                             bn2[l].reshape(1, NN), sum_e,
                             wg_x, wg_e, wg_u, bg1[l].reshape(1, -1),
                             Wg2[l], bg2[l].reshape(1, GN),
                             None, None, last, E)
        else:
            a_src_n = We1[l + 1][:NN]
            a_dst_n = We1[l + 1][NN:2 * NN]
            x, ps, pd, u2 = _node_tc(x, agg2, u2, wn_x, wn_a, wn_u,
                                     bn1[l].reshape(1, -1), Wn2[l],
                                     bn2[l].reshape(1, NN), sum_e,
                                     wg_x, wg_e, wg_u, bg1[l].reshape(1, -1),
                                     Wg2[l], bg2[l].reshape(1, GN),
                                     a_src_n, a_dst_n, last, E)

    return x, edge_attr[:, :EN], u2.reshape(GN)
